# baseline (speedup 1.0000x reference)
import jax
import jax.numpy as jnp
from jax import lax
from jax.experimental import pallas as pl

_NEG = -1e9
_K = 32


def _sortable(i32):
    return i32 ^ ((i32 >> 31) & jnp.int32(0x7FFFFFFF))


def _body(q_ref, k_ref, v_ref, s_ref, wq_ref, bq_ref, wo_ref, bo_ref, o_ref):
    qb = q_ref[0]
    kb = k_ref[0]

    scores = lax.dot_general(qb, kb, (((1,), (1,)), ((), ())),
                             preferred_element_type=jnp.float32)
    active = s_ref[0, 0] > 0.0
    scores = jnp.where(active[None, :], scores, _NEG)

    sx = _sortable(lax.bitcast_convert_type(scores, jnp.int32))
    minint = jnp.int32(-2**31)
    nrows = scores.shape[0]
    cand0 = jnp.zeros((nrows, 1), jnp.int32)
    bit0 = minint

    def step(_, carry):
        cand, bit = carry
        trial = cand | bit
        thr = trial ^ minint
        cnt = jnp.sum((sx >= thr).astype(jnp.int32), axis=1, keepdims=True)
        cand = jnp.where(cnt >= _K, trial, cand)
        return cand, lax.shift_right_logical(bit, 1)

    cand, _ = lax.fori_loop(0, 32, step, (cand0, bit0))
    thr_f = lax.bitcast_convert_type(_sortable(cand ^ minint), jnp.float32)
    topk_mask = scores >= thr_f

    qc = lax.dot_general(qb, wq_ref[0], (((1,), (0,)), ((), ())),
                         preferred_element_type=jnp.float32) + bq_ref[0]
    logits = lax.dot_general(qc, kb, (((1,), (1,)), ((), ())),
                             preferred_element_type=jnp.float32) + scores
    logits = jnp.where(topk_mask, logits, _NEG)

    m = jnp.max(logits, axis=1, keepdims=True)
    p = jnp.exp(logits - m)
    w = p / jnp.sum(p, axis=1, keepdims=True)

    out = lax.dot_general(w, v_ref[0], (((1,), (0,)), ((), ())),
                          preferred_element_type=jnp.float32)
    out = lax.dot_general(out, wo_ref[0], (((1,), (0,)), ((), ())),
                          preferred_element_type=jnp.float32) + bo_ref[0]
    o_ref[0] = out


def kernel(q, em_K, em_V, em_S, Wq, bq, Wo, bo):
    BSB_, NC_, D_ = q.shape
    M_ = em_K.shape[1]
    Bb = Wq.shape[0]

    s3 = em_S.reshape(BSB_, 1, M_)
    bq3 = bq.reshape(Bb, 1, D_)
    bo3 = bo.reshape(Bb, 1, D_)

    return pl.pallas_call(
        _body,
        grid=(BSB_,),
        in_specs=[
            pl.BlockSpec((1, NC_, D_), lambda b: (b, 0, 0)),
            pl.BlockSpec((1, M_, D_), lambda b: (b, 0, 0)),
            pl.BlockSpec((1, M_, D_), lambda b: (b, 0, 0)),
            pl.BlockSpec((1, 1, M_), lambda b: (b, 0, 0)),
            pl.BlockSpec((1, D_, D_), lambda b: (b % Bb, 0, 0)),
            pl.BlockSpec((1, 1, D_), lambda b: (b % Bb, 0, 0)),
            pl.BlockSpec((1, D_, D_), lambda b: (b % Bb, 0, 0)),
            pl.BlockSpec((1, 1, D_), lambda b: (b % Bb, 0, 0)),
        ],
        out_specs=pl.BlockSpec((1, NC_, D_), lambda b: (b, 0, 0)),
        out_shape=jax.ShapeDtypeStruct((BSB_, NC_, D_), jnp.float32),
    )(q, em_K, em_V, s3, Wq, bq3, Wo, bo3)
